# baseline (speedup 1.0000x reference)
import functools
import math

import jax
import jax.numpy as jnp
from jax import lax
from jax.experimental import pallas as pl
from jax.experimental.pallas import tpu as pltpu
from jax.experimental.pallas import tpu_sc as plsc

D_EMBED = 64
SCALE = math.sqrt(D_EMBED)
NUM_CORES = 2
NUM_SUBCORES = 16
NUM_WORKERS = NUM_CORES * NUM_SUBCORES
LANES = 16
NBUF = 4
LOOKAHEAD = 2
PREP_W = 1024
HALF = PREP_W // 2


def _prep_block(tab_t_ref, out_ref):
    block = tab_t_ref[...]
    row = jax.lax.broadcasted_iota(jnp.int32, (D_EMBED, D_EMBED), 0)
    col = jax.lax.broadcasted_iota(jnp.int32, (D_EMBED, D_EMBED), 1)
    eye_s = jnp.where(row == col, jnp.float32(SCALE), jnp.float32(0.0))

    def tr(m):
        return jax.lax.dot_general(
            m,
            eye_s,
            ((((0,), (0,))), ((), ())),
            preferred_element_type=jnp.float32,
        )

    lo = tr(block[:, :HALF])
    hi = tr(block[:, HALF:])
    out_ref[...] = jnp.concatenate([lo, hi], axis=1)


def _prep_table(tab_t):
    v = tab_t.shape[1]
    n_blocks = (v + PREP_W - 1) // PREP_W
    return pl.pallas_call(
        _prep_block,
        grid=(n_blocks,),
        in_specs=[pl.BlockSpec((D_EMBED, PREP_W), lambda i: (0, i))],
        out_specs=pl.BlockSpec((HALF, 2 * D_EMBED), lambda i: (i, 0)),
        out_shape=jax.ShapeDtypeStruct((n_blocks * HALF, 2 * D_EMBED), jnp.float32),
    )(tab_t)


def _build_sc_gather(xb: int, xt: int, v_pad: int):
    assert xb % (NUM_WORKERS * NBUF) == 0
    rows_per_worker = xb // NUM_WORKERS

    mesh = plsc.VectorSubcoreMesh(core_axis_name="c", subcore_axis_name="s")

    @functools.partial(
        pl.kernel,
        out_type=jax.ShapeDtypeStruct((xb, xt, D_EMBED), jnp.float32),
        mesh=mesh,
        scratch_types=[
            pltpu.VMEM((rows_per_worker, xt), jnp.int32),
            pltpu.VMEM((rows_per_worker, xt), jnp.int32),
            pltpu.VMEM((NBUF, xt, D_EMBED), jnp.float32),
            pltpu.SemaphoreType.DMA((NBUF,)),
            pltpu.SemaphoreType.DMA((NBUF,)),
        ],
        compiler_params=pltpu.CompilerParams(use_tc_tiling_on_sc=False),
    )
    def sc_gather(x_hbm, tab_hbm, out_hbm, idx_v, idxq_v, rows_v, gsem, ssem):
        wid = lax.axis_index("s") * NUM_CORES + lax.axis_index("c")
        base = wid * rows_per_worker
        pltpu.sync_copy(x_hbm.at[pl.ds(base, rows_per_worker)], idx_v)

        col_starts = [c * LANES for c in range(xt // LANES)]
        if xt % LANES:
            col_starts.append(xt - LANES)

        @pl.loop(0, rows_per_worker)
        def _remap(r):
            for c0 in col_starts:
                sl = pl.ds(c0, LANES)
                i = idx_v[r, sl]
                q = (i & ~(PREP_W - 1)) + ((i & (HALF - 1)) << 1) + (
                    (i >> 9) & 1
                )
                idxq_v[r, sl] = q

        def gather(r, b):
            return pltpu.make_async_copy(
                tab_hbm.at[idxq_v.at[r]],
                rows_v.at[b],
                gsem.at[b],
            )

        def store(r, b):
            return pltpu.make_async_copy(
                rows_v.at[b],
                out_hbm.at[base + r],
                ssem.at[b],
            )

        for r in range(LOOKAHEAD):
            gather(r, r).start()

        @pl.loop(0, rows_per_worker // NBUF)
        def _group(g):
            r0 = g * NBUF
            for b in range(NBUF):
                r = r0 + b
                b2 = (b + LOOKAHEAD) % NBUF

                gather(r, b).wait()

                @pl.when(r + LOOKAHEAD < rows_per_worker)
                def _start_next():
                    @pl.when(r + LOOKAHEAD >= NBUF)
                    def _drain_b2():
                        store(0, b2).wait()

                    gather(r + LOOKAHEAD, b2).start()

                store(r, b).start()

        for b in range(NBUF):
            store(0, b).wait()

    return sc_gather


def kernel(x, table):
    b, t = x.shape
    scaled2 = _prep_table(table.T)
    v_pad = scaled2.shape[0] * 2
    scaled = scaled2.reshape(v_pad, D_EMBED)
    return _build_sc_gather(b, t, v_pad)(x.astype(jnp.int32), scaled)


# speedup vs baseline: 1.4013x; 1.4013x over previous
import functools
import math

import jax
import jax.numpy as jnp
from jax import lax
from jax.experimental import pallas as pl
from jax.experimental.pallas import tpu as pltpu
from jax.experimental.pallas import tpu_sc as plsc

D_EMBED = 64
SCALE = math.sqrt(D_EMBED)
NUM_CORES = 2
NUM_SUBCORES = 16
NUM_WORKERS = NUM_CORES * NUM_SUBCORES
LANES = 16
NBUF = 4
LOOKAHEAD = 2
PREP_W = 4096
HALF = PREP_W // 2
HALF_SHIFT = HALF.bit_length() - 1


def _prep_block(tab_t_ref, out_ref):
    block = tab_t_ref[...]
    row = jax.lax.broadcasted_iota(jnp.int32, (D_EMBED, D_EMBED), 0)
    col = jax.lax.broadcasted_iota(jnp.int32, (D_EMBED, D_EMBED), 1)
    eye_s = jnp.where(row == col, jnp.float32(SCALE), jnp.float32(0.0))

    def tr(m):
        return jax.lax.dot_general(
            m,
            eye_s,
            ((((0,), (0,))), ((), ())),
            preferred_element_type=jnp.float32,
        )

    lo = tr(block[:, :HALF])
    hi = tr(block[:, HALF:])
    out_ref[...] = jnp.concatenate([lo, hi], axis=1)


def _prep_table(tab_t):
    v = tab_t.shape[1]
    n_blocks = (v + PREP_W - 1) // PREP_W
    return pl.pallas_call(
        _prep_block,
        grid=(n_blocks,),
        in_specs=[pl.BlockSpec((D_EMBED, PREP_W), lambda i: (0, i))],
        out_specs=pl.BlockSpec((HALF, 2 * D_EMBED), lambda i: (i, 0)),
        out_shape=jax.ShapeDtypeStruct((n_blocks * HALF, 2 * D_EMBED), jnp.float32),
    )(tab_t)


def _build_sc_gather(xb: int, xt: int, v_pad: int):
    assert xb % (NUM_WORKERS * NBUF) == 0
    rows_per_worker = xb // NUM_WORKERS

    mesh = plsc.VectorSubcoreMesh(core_axis_name="c", subcore_axis_name="s")

    @functools.partial(
        pl.kernel,
        out_type=jax.ShapeDtypeStruct((xb, xt, D_EMBED), jnp.float32),
        mesh=mesh,
        scratch_types=[
            pltpu.VMEM((rows_per_worker, xt), jnp.int32),
            pltpu.VMEM((rows_per_worker, xt), jnp.int32),
            pltpu.VMEM((NBUF, xt, D_EMBED), jnp.float32),
            pltpu.SemaphoreType.DMA((NBUF,)),
            pltpu.SemaphoreType.DMA((NBUF,)),
        ],
        compiler_params=pltpu.CompilerParams(use_tc_tiling_on_sc=False),
    )
    def sc_gather(x_hbm, tab_hbm, out_hbm, idx_v, idxq_v, rows_v, gsem, ssem):
        wid = lax.axis_index("s") * NUM_CORES + lax.axis_index("c")
        base = wid * rows_per_worker
        pltpu.sync_copy(x_hbm.at[pl.ds(base, rows_per_worker)], idx_v)

        col_starts = [c * LANES for c in range(xt // LANES)]
        if xt % LANES:
            col_starts.append(xt - LANES)

        @pl.loop(0, rows_per_worker)
        def _remap(r):
            for c0 in col_starts:
                sl = pl.ds(c0, LANES)
                i = idx_v[r, sl]
                q = (i & ~(PREP_W - 1)) + ((i & (HALF - 1)) << 1) + (
                    (i >> HALF_SHIFT) & 1
                )
                idxq_v[r, sl] = q

        def gather(r, b):
            return pltpu.make_async_copy(
                tab_hbm.at[idxq_v.at[r]],
                rows_v.at[b],
                gsem.at[b],
            )

        def store(r, b):
            return pltpu.make_async_copy(
                rows_v.at[b],
                out_hbm.at[base + r],
                ssem.at[b],
            )

        for r in range(LOOKAHEAD):
            gather(r, r).start()

        @pl.loop(0, rows_per_worker // NBUF)
        def _group(g):
            r0 = g * NBUF
            for b in range(NBUF):
                r = r0 + b
                b2 = (b + LOOKAHEAD) % NBUF

                gather(r, b).wait()

                @pl.when(r + LOOKAHEAD < rows_per_worker)
                def _start_next():
                    @pl.when(r + LOOKAHEAD >= NBUF)
                    def _drain_b2():
                        store(0, b2).wait()

                    gather(r + LOOKAHEAD, b2).start()

                store(r, b).start()

        for b in range(NBUF):
            store(0, b).wait()

    return sc_gather


def kernel(x, table):
    b, t = x.shape
    scaled2 = _prep_table(table.T)
    v_pad = scaled2.shape[0] * 2
    scaled = scaled2.reshape(-1).reshape(v_pad, D_EMBED)
    return _build_sc_gather(b, t, v_pad)(x.astype(jnp.int32), scaled)


# speedup vs baseline: 2.0365x; 1.4533x over previous
import functools
import math

import jax
import jax.numpy as jnp
from jax import lax
from jax.experimental import pallas as pl
from jax.experimental.pallas import tpu as pltpu
from jax.experimental.pallas import tpu_sc as plsc

D_EMBED = 64
SCALE = math.sqrt(D_EMBED)
NUM_CORES = 2
NUM_SUBCORES = 16
NUM_WORKERS = NUM_CORES * NUM_SUBCORES
LANES = 16
NBUF = 4
LOOKAHEAD = 2
PREP_W = 4096
HALF = PREP_W // 2
HALF_SHIFT = HALF.bit_length() - 1
OUT_TBLK = 2


def _eye(n, scale):
    row = jax.lax.broadcasted_iota(jnp.int32, (n, n), 0)
    col = jax.lax.broadcasted_iota(jnp.int32, (n, n), 1)
    return jnp.where(row == col, jnp.float32(scale), jnp.float32(0.0))


def _tr(m, eye):
    return jax.lax.dot_general(
        m, eye, ((((0,), (0,))), ((), ())), preferred_element_type=jnp.float32
    )


def _trr(eye, m):
    return jax.lax.dot_general(
        eye, m, ((((1,), (1,))), ((), ())), preferred_element_type=jnp.float32
    )


def _prep_block(tab_t_ref, out_ref):
    block = tab_t_ref[...]
    eye_s = _eye(D_EMBED, SCALE)
    lo = _tr(block[:, :HALF], eye_s)
    hi = _tr(block[:, HALF:], eye_s)
    out_ref[...] = jnp.concatenate([lo, hi], axis=1)


def _prep_table(tab_t):
    v = tab_t.shape[1]
    n_blocks = (v + PREP_W - 1) // PREP_W
    return pl.pallas_call(
        _prep_block,
        grid=(n_blocks,),
        in_specs=[pl.BlockSpec((D_EMBED, PREP_W), lambda i: (0, i))],
        out_specs=pl.BlockSpec((HALF, 2 * D_EMBED), lambda i: (i, 0)),
        out_shape=jax.ShapeDtypeStruct((n_blocks * HALF, 2 * D_EMBED), jnp.float32),
    )(tab_t)


def _out_block(in_ref, out_ref):
    eye1 = _eye(D_EMBED, 1.0)
    for j in range(OUT_TBLK):
        slab = in_ref[pl.ds(j * 2048, 2048), :]
        lo = _trr(eye1, slab[:, :D_EMBED])
        hi = _trr(eye1, slab[:, D_EMBED:])
        out_ref[j] = jnp.concatenate([lo, hi], axis=1)


def _out_transpose(g2, xb, xt):
    rows_per_t = xb * D_EMBED // 128
    return pl.pallas_call(
        _out_block,
        grid=(xt // OUT_TBLK,),
        in_specs=[
            pl.BlockSpec((OUT_TBLK * rows_per_t, 128), lambda i: (i, 0))
        ],
        out_specs=pl.BlockSpec((OUT_TBLK, D_EMBED, xb), lambda i: (i, 0, 0)),
        out_shape=jax.ShapeDtypeStruct((xt, D_EMBED, xb), jnp.float32),
    )(g2)


def _build_sc_gather(xb: int, xt: int, v_pad: int):
    assert xb % (2 * NUM_WORKERS * D_EMBED) == 0
    hb = xb // 2
    per_w = xb // NUM_WORKERS

    mesh = plsc.VectorSubcoreMesh(core_axis_name="c", subcore_axis_name="s")

    @functools.partial(
        pl.kernel,
        out_type=jax.ShapeDtypeStruct((xt, xb, D_EMBED), jnp.float32),
        mesh=mesh,
        scratch_types=[
            pltpu.VMEM((2, xt, per_w // 2), jnp.int32),
            pltpu.VMEM((xt, per_w), jnp.int32),
            pltpu.VMEM((NBUF, per_w, D_EMBED), jnp.float32),
            pltpu.SemaphoreType.DMA((NBUF,)),
            pltpu.SemaphoreType.DMA((NBUF,)),
        ],
        compiler_params=pltpu.CompilerParams(
            use_tc_tiling_on_sc=False, needs_layout_passes=False
        ),
    )
    def sc_gather(xt_hbm, tab_hbm, out_hbm, idx_v, idxq_v, rows_v, gsem, ssem):
        wid = lax.axis_index("s") * NUM_CORES + lax.axis_index("c")
        a0 = wid * (per_w // 2)
        pltpu.sync_copy(
            xt_hbm.at[pl.ds(0, xt), pl.ds(a0, per_w // 2)], idx_v.at[0]
        )
        pltpu.sync_copy(
            xt_hbm.at[pl.ds(0, xt), pl.ds(hb + a0, per_w // 2)], idx_v.at[1]
        )

        @pl.loop(0, xt)
        def _remap(t):
            tvec = jnp.broadcast_to(t, (LANES,)).astype(jnp.int32)
            for c0 in range(0, per_w, LANES):
                jj = jax.lax.iota(jnp.int32, LANES) + c0
                i = plsc.load_gather(idx_v, [jj & 1, tvec, jj >> 1])
                q = (i & ~(PREP_W - 1)) + ((i & (HALF - 1)) << 1) + (
                    (i >> HALF_SHIFT) & 1
                )
                idxq_v[t, pl.ds(c0, LANES)] = q

        def gather(t, b):
            return pltpu.make_async_copy(
                tab_hbm.at[idxq_v.at[t]], rows_v.at[b], gsem.at[b]
            )

        def store(t, b):
            return pltpu.make_async_copy(
                rows_v.at[b],
                out_hbm.at[t, pl.ds(wid * per_w, per_w)],
                ssem.at[b],
            )

        for t in range(LOOKAHEAD):
            gather(t, t).start()

        @pl.loop(0, xt // NBUF)
        def _group(g):
            t0 = g * NBUF
            for b in range(NBUF):
                t = t0 + b
                b2 = (b + LOOKAHEAD) % NBUF

                gather(t, b).wait()

                @pl.when(t + LOOKAHEAD < xt)
                def _start_next():
                    @pl.when(t + LOOKAHEAD >= NBUF)
                    def _drain_b2():
                        store(0, b2).wait()

                    gather(t + LOOKAHEAD, b2).start()

                store(t, b).start()

        for b in range(NBUF):
            store(0, b).wait()

    return sc_gather


def kernel(x, table):
    b, t = x.shape
    scaled2 = _prep_table(table.T)
    v_pad = scaled2.shape[0] * 2
    scaled = scaled2.reshape(-1).reshape(v_pad, D_EMBED)
    out_sc = _build_sc_gather(b, t, v_pad)(x.T.astype(jnp.int32), scaled)
    g2 = out_sc.reshape(-1).reshape(t * b * D_EMBED // 128, 128)
    out_t = _out_transpose(g2, b, t)
    return jnp.transpose(out_t, (2, 0, 1))


# speedup vs baseline: 2.3708x; 1.1642x over previous
import functools
import math

import jax
import jax.numpy as jnp
from jax import lax
from jax.experimental import pallas as pl
from jax.experimental.pallas import tpu as pltpu
from jax.experimental.pallas import tpu_sc as plsc

D_EMBED = 64
SCALE = math.sqrt(D_EMBED)
NUM_CORES = 2
NUM_SUBCORES = 16
NUM_WORKERS = NUM_CORES * NUM_SUBCORES
LANES = 16
NBUF = 4
LOOKAHEAD = 2
PREP_W = 8192
HALF = PREP_W // 2
HALF_SHIFT = HALF.bit_length() - 1
OUT_TBLK = 4


def _eye(n, scale):
    row = jax.lax.broadcasted_iota(jnp.int32, (n, n), 0)
    col = jax.lax.broadcasted_iota(jnp.int32, (n, n), 1)
    return jnp.where(row == col, jnp.float32(scale), jnp.float32(0.0))


def _tr(m, eye):
    return jax.lax.dot_general(
        m, eye, ((((0,), (0,))), ((), ())), preferred_element_type=jnp.float32
    )


def _trr(eye, m):
    return jax.lax.dot_general(
        eye, m, ((((1,), (1,))), ((), ())), preferred_element_type=jnp.float32
    )


def _prep_block(tab_t_ref, out_ref):
    block = tab_t_ref[...]
    eye_s = _eye(D_EMBED, SCALE)
    lo = _tr(block[:, :HALF], eye_s)
    hi = _tr(block[:, HALF:], eye_s)
    out_ref[...] = jnp.concatenate([lo, hi], axis=1)


def _prep_table(tab_t):
    v = tab_t.shape[1]
    n_blocks = (v + PREP_W - 1) // PREP_W
    return pl.pallas_call(
        _prep_block,
        grid=(n_blocks,),
        in_specs=[pl.BlockSpec((D_EMBED, PREP_W), lambda i: (0, i))],
        out_specs=pl.BlockSpec((HALF, 2 * D_EMBED), lambda i: (i, 0)),
        out_shape=jax.ShapeDtypeStruct((n_blocks * HALF, 2 * D_EMBED), jnp.float32),
    )(tab_t)


def _out_block(in_ref, out_ref):
    eye1 = _eye(D_EMBED, 1.0)
    for j in range(OUT_TBLK):
        slab = in_ref[pl.ds(j * 2048, 2048), :]
        lo = _trr(eye1, slab[:, :D_EMBED])
        hi = _trr(eye1, slab[:, D_EMBED:])
        out_ref[j] = jnp.concatenate([lo, hi], axis=1)


def _out_transpose(g2, xb, xt):
    rows_per_t = xb * D_EMBED // 128
    return pl.pallas_call(
        _out_block,
        grid=(xt // OUT_TBLK,),
        in_specs=[
            pl.BlockSpec((OUT_TBLK * rows_per_t, 128), lambda i: (i, 0))
        ],
        out_specs=pl.BlockSpec((OUT_TBLK, D_EMBED, xb), lambda i: (i, 0, 0)),
        out_shape=jax.ShapeDtypeStruct((xt, D_EMBED, xb), jnp.float32),
    )(g2)


def _build_sc_gather(xb: int, xt: int, v_pad: int):
    assert xb % (2 * NUM_WORKERS * D_EMBED) == 0
    hb = xb // 2
    per_w = xb // NUM_WORKERS

    mesh = plsc.VectorSubcoreMesh(core_axis_name="c", subcore_axis_name="s")

    @functools.partial(
        pl.kernel,
        out_type=jax.ShapeDtypeStruct((xt, xb, D_EMBED), jnp.float32),
        mesh=mesh,
        scratch_types=[
            pltpu.VMEM((2, xt, per_w // 2), jnp.int32),
            pltpu.VMEM((xt, per_w), jnp.int32),
            pltpu.VMEM((NBUF, per_w, D_EMBED), jnp.float32),
            pltpu.SemaphoreType.DMA((NBUF,)),
            pltpu.SemaphoreType.DMA((NBUF,)),
        ],
        compiler_params=pltpu.CompilerParams(
            use_tc_tiling_on_sc=False, needs_layout_passes=False
        ),
    )
    def sc_gather(xt_hbm, tab_hbm, out_hbm, idx_v, idxq_v, rows_v, gsem, ssem):
        wid = lax.axis_index("s") * NUM_CORES + lax.axis_index("c")
        a0 = wid * (per_w // 2)
        pltpu.sync_copy(
            xt_hbm.at[pl.ds(0, xt), pl.ds(a0, per_w // 2)], idx_v.at[0]
        )
        pltpu.sync_copy(
            xt_hbm.at[pl.ds(0, xt), pl.ds(hb + a0, per_w // 2)], idx_v.at[1]
        )

        @pl.loop(0, xt)
        def _remap(t):
            tvec = jnp.broadcast_to(t, (LANES,)).astype(jnp.int32)
            for c0 in range(0, per_w, LANES):
                jj = jax.lax.iota(jnp.int32, LANES) + c0
                i = plsc.load_gather(idx_v, [jj & 1, tvec, jj >> 1])
                q = (i & ~(PREP_W - 1)) + ((i & (HALF - 1)) << 1) + (
                    (i >> HALF_SHIFT) & 1
                )
                idxq_v[t, pl.ds(c0, LANES)] = q

        def gather(t, b):
            return pltpu.make_async_copy(
                tab_hbm.at[idxq_v.at[t]], rows_v.at[b], gsem.at[b]
            )

        def store(t, b):
            return pltpu.make_async_copy(
                rows_v.at[b],
                out_hbm.at[t, pl.ds(wid * per_w, per_w)],
                ssem.at[b],
            )

        for t in range(LOOKAHEAD):
            gather(t, t).start()

        @pl.loop(0, xt // NBUF)
        def _group(g):
            t0 = g * NBUF
            for b in range(NBUF):
                t = t0 + b
                b2 = (b + LOOKAHEAD) % NBUF

                gather(t, b).wait()

                @pl.when(t + LOOKAHEAD < xt)
                def _start_next():
                    @pl.when(t + LOOKAHEAD >= NBUF)
                    def _drain_b2():
                        store(0, b2).wait()

                    gather(t + LOOKAHEAD, b2).start()

                store(t, b).start()

        for b in range(NBUF):
            store(0, b).wait()

    return sc_gather


def kernel(x, table):
    b, t = x.shape
    scaled2 = _prep_table(table.T)
    v_pad = scaled2.shape[0] * 2
    scaled = scaled2.reshape(-1).reshape(v_pad, D_EMBED)
    out_sc = _build_sc_gather(b, t, v_pad)(x.T.astype(jnp.int32), scaled)
    g2 = out_sc.reshape(-1).reshape(t * b * D_EMBED // 128, 128)
    out_t = _out_transpose(g2, b, t)
    return jnp.transpose(out_t, (2, 0, 1))


# speedup vs baseline: 2.5505x; 1.0758x over previous
import functools
import math

import jax
import jax.numpy as jnp
from jax import lax
from jax.experimental import pallas as pl
from jax.experimental.pallas import tpu as pltpu
from jax.experimental.pallas import tpu_sc as plsc

D_EMBED = 64
SCALE = math.sqrt(D_EMBED)
NUM_CORES = 2
NUM_SUBCORES = 16
NUM_WORKERS = NUM_CORES * NUM_SUBCORES
LANES = 16
NBUF = 4
LOOKAHEAD = 2
PREP_W = 16384
HALF = PREP_W // 2
HALF_SHIFT = HALF.bit_length() - 1
OUT_TBLK = 8


def _eye(n, scale):
    row = jax.lax.broadcasted_iota(jnp.int32, (n, n), 0)
    col = jax.lax.broadcasted_iota(jnp.int32, (n, n), 1)
    return jnp.where(row == col, jnp.float32(scale), jnp.float32(0.0))


def _tr(m, eye):
    return jax.lax.dot_general(
        m, eye, ((((0,), (0,))), ((), ())), preferred_element_type=jnp.float32
    )


def _trr(eye, m):
    return jax.lax.dot_general(
        eye, m, ((((1,), (1,))), ((), ())), preferred_element_type=jnp.float32
    )


def _prep_block(tab_t_ref, out_ref):
    block = tab_t_ref[...]
    eye_s = _eye(D_EMBED, SCALE)
    lo = _tr(block[:, :HALF], eye_s)
    hi = _tr(block[:, HALF:], eye_s)
    out_ref[...] = jnp.concatenate([lo, hi], axis=1)


def _prep_table(tab_t):
    v = tab_t.shape[1]
    n_blocks = (v + PREP_W - 1) // PREP_W
    return pl.pallas_call(
        _prep_block,
        grid=(n_blocks,),
        in_specs=[pl.BlockSpec((D_EMBED, PREP_W), lambda i: (0, i))],
        out_specs=pl.BlockSpec((HALF, 2 * D_EMBED), lambda i: (i, 0)),
        out_shape=jax.ShapeDtypeStruct((n_blocks * HALF, 2 * D_EMBED), jnp.float32),
    )(tab_t)


def _out_block(in_ref, out_ref):
    eye1 = _eye(D_EMBED, 1.0)
    for j in range(OUT_TBLK):
        slab = in_ref[pl.ds(j * 2048, 2048), :]
        lo = _trr(eye1, slab[:, :D_EMBED])
        hi = _trr(eye1, slab[:, D_EMBED:])
        out_ref[j] = jnp.concatenate([lo, hi], axis=1)


def _out_transpose(g2, xb, xt):
    rows_per_t = xb * D_EMBED // 128
    return pl.pallas_call(
        _out_block,
        grid=(xt // OUT_TBLK,),
        in_specs=[
            pl.BlockSpec((OUT_TBLK * rows_per_t, 128), lambda i: (i, 0))
        ],
        out_specs=pl.BlockSpec((OUT_TBLK, D_EMBED, xb), lambda i: (i, 0, 0)),
        out_shape=jax.ShapeDtypeStruct((xt, D_EMBED, xb), jnp.float32),
    )(g2)


def _build_sc_gather(xb: int, xt: int, v_pad: int):
    assert xb % (2 * NUM_WORKERS * D_EMBED) == 0
    hb = xb // 2
    per_w = xb // NUM_WORKERS

    mesh = plsc.VectorSubcoreMesh(core_axis_name="c", subcore_axis_name="s")

    @functools.partial(
        pl.kernel,
        out_type=jax.ShapeDtypeStruct((xt, xb, D_EMBED), jnp.float32),
        mesh=mesh,
        scratch_types=[
            pltpu.VMEM((2, xt, per_w // 2), jnp.int32),
            pltpu.VMEM((xt, per_w), jnp.int32),
            pltpu.VMEM((NBUF, per_w, D_EMBED), jnp.float32),
            pltpu.SemaphoreType.DMA((NBUF,)),
            pltpu.SemaphoreType.DMA((NBUF,)),
        ],
        compiler_params=pltpu.CompilerParams(
            use_tc_tiling_on_sc=False, needs_layout_passes=False
        ),
    )
    def sc_gather(xt_hbm, tab_hbm, out_hbm, idx_v, idxq_v, rows_v, gsem, ssem):
        wid = lax.axis_index("s") * NUM_CORES + lax.axis_index("c")
        a0 = wid * (per_w // 2)
        pltpu.sync_copy(
            xt_hbm.at[pl.ds(0, xt), pl.ds(a0, per_w // 2)], idx_v.at[0]
        )
        pltpu.sync_copy(
            xt_hbm.at[pl.ds(0, xt), pl.ds(hb + a0, per_w // 2)], idx_v.at[1]
        )

        @pl.loop(0, xt)
        def _remap(t):
            tvec = jnp.broadcast_to(t, (LANES,)).astype(jnp.int32)
            for c0 in range(0, per_w, LANES):
                jj = jax.lax.iota(jnp.int32, LANES) + c0
                i = plsc.load_gather(idx_v, [jj & 1, tvec, jj >> 1])
                q = (i & ~(PREP_W - 1)) + ((i & (HALF - 1)) << 1) + (
                    (i >> HALF_SHIFT) & 1
                )
                idxq_v[t, pl.ds(c0, LANES)] = q

        def gather(t, b):
            return pltpu.make_async_copy(
                tab_hbm.at[idxq_v.at[t]], rows_v.at[b], gsem.at[b]
            )

        def store(t, b):
            return pltpu.make_async_copy(
                rows_v.at[b],
                out_hbm.at[t, pl.ds(wid * per_w, per_w)],
                ssem.at[b],
            )

        for t in range(LOOKAHEAD):
            gather(t, t).start()

        @pl.loop(0, xt // NBUF)
        def _group(g):
            t0 = g * NBUF
            for b in range(NBUF):
                t = t0 + b
                b2 = (b + LOOKAHEAD) % NBUF

                gather(t, b).wait()

                @pl.when(t + LOOKAHEAD < xt)
                def _start_next():
                    @pl.when(t + LOOKAHEAD >= NBUF)
                    def _drain_b2():
                        store(0, b2).wait()

                    gather(t + LOOKAHEAD, b2).start()

                store(t, b).start()

        for b in range(NBUF):
            store(0, b).wait()

    return sc_gather


def kernel(x, table):
    b, t = x.shape
    scaled2 = _prep_table(table.T)
    v_pad = scaled2.shape[0] * 2
    scaled = scaled2.reshape(-1).reshape(v_pad, D_EMBED)
    out_sc = _build_sc_gather(b, t, v_pad)(x.T.astype(jnp.int32), scaled)
    g2 = out_sc.reshape(-1).reshape(t * b * D_EMBED // 128, 128)
    out_t = _out_transpose(g2, b, t)
    return jnp.transpose(out_t, (2, 0, 1))


# speedup vs baseline: 2.6182x; 1.0266x over previous
import functools
import math

import jax
import jax.numpy as jnp
from jax import lax
from jax.experimental import pallas as pl
from jax.experimental.pallas import tpu as pltpu
from jax.experimental.pallas import tpu_sc as plsc

D_EMBED = 64
SCALE = math.sqrt(D_EMBED)
NUM_CORES = 2
NUM_SUBCORES = 16
NUM_WORKERS = NUM_CORES * NUM_SUBCORES
LANES = 16
NBUF = 4
LOOKAHEAD = 2
PREP_W = 32768
HALF = PREP_W // 2
HALF_SHIFT = HALF.bit_length() - 1
OUT_TBLK = 10


def _eye(n, scale):
    row = jax.lax.broadcasted_iota(jnp.int32, (n, n), 0)
    col = jax.lax.broadcasted_iota(jnp.int32, (n, n), 1)
    return jnp.where(row == col, jnp.float32(scale), jnp.float32(0.0))


def _tr(m, eye):
    return jax.lax.dot_general(
        m, eye, ((((0,), (0,))), ((), ())), preferred_element_type=jnp.float32
    )


def _trr(eye, m):
    return jax.lax.dot_general(
        eye, m, ((((1,), (1,))), ((), ())), preferred_element_type=jnp.float32
    )


def _prep_block(tab_t_ref, out_ref):
    block = tab_t_ref[...]
    eye_s = _eye(D_EMBED, SCALE)
    lo = _tr(block[:, :HALF], eye_s)
    hi = _tr(block[:, HALF:], eye_s)
    out_ref[...] = jnp.concatenate([lo, hi], axis=1)


def _prep_table(tab_t):
    v = tab_t.shape[1]
    n_blocks = (v + PREP_W - 1) // PREP_W
    return pl.pallas_call(
        _prep_block,
        grid=(n_blocks,),
        in_specs=[pl.BlockSpec((D_EMBED, PREP_W), lambda i: (0, i))],
        out_specs=pl.BlockSpec((HALF, 2 * D_EMBED), lambda i: (i, 0)),
        out_shape=jax.ShapeDtypeStruct((n_blocks * HALF, 2 * D_EMBED), jnp.float32),
    )(tab_t)


def _out_block(in_ref, out_ref):
    eye1 = _eye(D_EMBED, 1.0)
    for j in range(OUT_TBLK):
        slab = in_ref[pl.ds(j * 2048, 2048), :]
        lo = _trr(eye1, slab[:, :D_EMBED])
        hi = _trr(eye1, slab[:, D_EMBED:])
        out_ref[j] = jnp.concatenate([lo, hi], axis=1)


def _out_transpose(g2, xb, xt):
    rows_per_t = xb * D_EMBED // 128
    return pl.pallas_call(
        _out_block,
        grid=(xt // OUT_TBLK,),
        in_specs=[
            pl.BlockSpec((OUT_TBLK * rows_per_t, 128), lambda i: (i, 0))
        ],
        out_specs=pl.BlockSpec((OUT_TBLK, D_EMBED, xb), lambda i: (i, 0, 0)),
        out_shape=jax.ShapeDtypeStruct((xt, D_EMBED, xb), jnp.float32),
    )(g2)


def _build_sc_gather(xb: int, xt: int, v_pad: int):
    assert xb % (2 * NUM_WORKERS * D_EMBED) == 0
    hb = xb // 2
    per_w = xb // NUM_WORKERS

    mesh = plsc.VectorSubcoreMesh(core_axis_name="c", subcore_axis_name="s")

    @functools.partial(
        pl.kernel,
        out_type=jax.ShapeDtypeStruct((xt, xb, D_EMBED), jnp.float32),
        mesh=mesh,
        scratch_types=[
            pltpu.VMEM((2, xt, per_w // 2), jnp.int32),
            pltpu.VMEM((xt, per_w), jnp.int32),
            pltpu.VMEM((NBUF, per_w, D_EMBED), jnp.float32),
            pltpu.SemaphoreType.DMA((NBUF,)),
            pltpu.SemaphoreType.DMA((NBUF,)),
        ],
        compiler_params=pltpu.CompilerParams(
            use_tc_tiling_on_sc=False, needs_layout_passes=False
        ),
    )
    def sc_gather(xt_hbm, tab_hbm, out_hbm, idx_v, idxq_v, rows_v, gsem, ssem):
        wid = lax.axis_index("s") * NUM_CORES + lax.axis_index("c")
        a0 = wid * (per_w // 2)
        pltpu.sync_copy(
            xt_hbm.at[pl.ds(0, xt), pl.ds(a0, per_w // 2)], idx_v.at[0]
        )
        pltpu.sync_copy(
            xt_hbm.at[pl.ds(0, xt), pl.ds(hb + a0, per_w // 2)], idx_v.at[1]
        )

        @pl.loop(0, xt)
        def _remap(t):
            tvec = jnp.broadcast_to(t, (LANES,)).astype(jnp.int32)
            for c0 in range(0, per_w, LANES):
                jj = jax.lax.iota(jnp.int32, LANES) + c0
                i = plsc.load_gather(idx_v, [jj & 1, tvec, jj >> 1])
                q = (i & ~(PREP_W - 1)) + ((i & (HALF - 1)) << 1) + (
                    (i >> HALF_SHIFT) & 1
                )
                idxq_v[t, pl.ds(c0, LANES)] = q

        def gather(t, b):
            return pltpu.make_async_copy(
                tab_hbm.at[idxq_v.at[t]], rows_v.at[b], gsem.at[b]
            )

        def store(t, b):
            return pltpu.make_async_copy(
                rows_v.at[b],
                out_hbm.at[t, pl.ds(wid * per_w, per_w)],
                ssem.at[b],
            )

        for t in range(LOOKAHEAD):
            gather(t, t).start()

        @pl.loop(0, xt // NBUF)
        def _group(g):
            t0 = g * NBUF
            for b in range(NBUF):
                t = t0 + b
                b2 = (b + LOOKAHEAD) % NBUF

                gather(t, b).wait()

                @pl.when(t + LOOKAHEAD < xt)
                def _start_next():
                    @pl.when(t + LOOKAHEAD >= NBUF)
                    def _drain_b2():
                        store(0, b2).wait()

                    gather(t + LOOKAHEAD, b2).start()

                store(t, b).start()

        for b in range(NBUF):
            store(0, b).wait()

    return sc_gather


def kernel(x, table):
    b, t = x.shape
    scaled2 = _prep_table(table.T)
    v_pad = scaled2.shape[0] * 2
    scaled = scaled2.reshape(-1).reshape(v_pad, D_EMBED)
    out_sc = _build_sc_gather(b, t, v_pad)(x.T.astype(jnp.int32), scaled)
    g2 = out_sc.reshape(-1).reshape(t * b * D_EMBED // 128, 128)
    out_t = _out_transpose(g2, b, t)
    return jnp.transpose(out_t, (2, 0, 1))
